# baseline (speedup 1.0000x reference)
import functools

import jax
import jax.numpy as jnp
from jax import lax
from jax.experimental import pallas as pl
from jax.experimental.pallas import tpu as pltpu
from jax.experimental.pallas import tpu_sc as plsc

N_NODES = 10000
D = 128
N_EDGES = 320000
NC, NS = 2, 16
NW = NC * NS
PAD_N = 10240
ROWS_PER_TILE = PAD_N // NS
CH = 128
NCHT = 79
E_PAD = NW * NCHT * CH
NCH_TOTAL = NW * NCHT
NCHT_C0 = 156
NCHT_C1 = 2 * NCHT - NCHT_C0

_mesh = plsc.VectorSubcoreMesh(core_axis_name="c", subcore_axis_name="s")


@functools.partial(
    pl.kernel,
    out_type=jax.ShapeDtypeStruct((NC, PAD_N), jnp.float32),
    mesh=_mesh,
    scratch_types=[
        pltpu.VMEM((NCHT, 2, CH), jnp.int32),
        pltpu.VMEM((CH,), jnp.float32),
        pltpu.VMEM((ROWS_PER_TILE,), jnp.float32),
        pltpu.VMEM_SHARED((PAD_N,), jnp.float32),
        pltpu.SemaphoreType.DMA,
    ],
)
def _deg_kernel(idx_hbm, degp_hbm, idx_v, ones_v, zero_v, acc_sh, sem):
    c = lax.axis_index("c")
    s = lax.axis_index("s")
    w = s * NC + c

    for i in range(CH // 16):
        ones_v[pl.ds(i * 16, 16)] = jnp.ones((16,), jnp.float32)

    def _zfill(i, carry):
        zero_v[pl.ds(i * 16, 16)] = jnp.zeros((16,), jnp.float32)
        return carry

    lax.fori_loop(0, ROWS_PER_TILE // 16, _zfill, 0)
    pltpu.sync_copy(idx_hbm.at[pl.ds(w * NCHT, NCHT)], idx_v)
    pltpu.sync_copy(zero_v, acc_sh.at[pl.ds(s * ROWS_PER_TILE, ROWS_PER_TILE)])
    plsc.subcore_barrier()

    GRP = 16

    def _group(g, carry):
        for k in range(GRP):
            @pl.when(g * GRP + k < NCHT)
            def _():
                pltpu.async_copy(
                    ones_v, acc_sh.at[idx_v.at[g * GRP + k, 1]], sem,
                    add=True)
        for k in range(GRP):
            @pl.when(g * GRP + k < NCHT)
            def _():
                pltpu.make_async_copy(
                    ones_v, acc_sh.at[idx_v.at[g * GRP + k, 1]], sem).wait()
        return carry

    lax.fori_loop(0, (NCHT + GRP - 1) // GRP, _group, 0)
    plsc.subcore_barrier()
    pltpu.sync_copy(acc_sh.at[pl.ds(s * ROWS_PER_TILE, ROWS_PER_TILE)],
                    degp_hbm.at[c, pl.ds(s * ROWS_PER_TILE, ROWS_PER_TILE)])


@functools.partial(
    pl.kernel,
    out_type=jax.ShapeDtypeStruct((NC, PAD_N, D), jnp.float32),
    mesh=_mesh,
    scratch_types=[
        pltpu.VMEM((3, 2, CH), jnp.int32),
        pltpu.VMEM((2, CH, D), jnp.float32),
        pltpu.VMEM((16, D), jnp.float32),
        pltpu.VMEM_SHARED((PAD_N, D), jnp.float32),
        pltpu.SemaphoreType.DMA((3,)),
        pltpu.SemaphoreType.DMA((2,)),
        pltpu.SemaphoreType.DMA((2,)),
    ],
)
def _msg_kernel(idx_hbm, h_hbm, out_hbm,
                idx_v, rows_v, zero_v, acc_sh, isem, gsem, ssem):
    c = lax.axis_index("c")
    s = lax.axis_index("s")
    start = jnp.where(c == 0, s * NCHT_C0, NS * NCHT_C0 + s * NCHT_C1)
    cnt = jnp.where(c == 0, NCHT_C0, NCHT_C1)

    for r in range(16):
        for k in range(D // 16):
            zero_v[r, pl.ds(k * 16, 16)] = jnp.zeros((16,), jnp.float32)

    def _zfill(i, carry):
        pltpu.sync_copy(
            zero_v, acc_sh.at[pl.ds(s * ROWS_PER_TILE + i * 16, 16), :])
        return carry

    lax.fori_loop(0, ROWS_PER_TILE // 16, _zfill, 0)
    plsc.subcore_barrier()

    def _start_idx(j):
        pltpu.async_copy(idx_hbm.at[start + j], idx_v.at[j % 3],
                         isem.at[j % 3])

    def _wait_idx(j):
        pltpu.make_async_copy(
            idx_hbm.at[start + j], idx_v.at[j % 3], isem.at[j % 3]).wait()

    def _start_gather(j):
        pltpu.async_copy(
            h_hbm.at[idx_v.at[j % 3, 0]], rows_v.at[j & 1], gsem.at[j & 1])

    def _wait_gather(j):
        pltpu.make_async_copy(
            h_hbm.at[idx_v.at[j % 3, 0]], rows_v.at[j & 1],
            gsem.at[j & 1]).wait()

    def _start_scatter(j):
        pltpu.async_copy(
            rows_v.at[j & 1], acc_sh.at[idx_v.at[j % 3, 1]], ssem.at[j & 1],
            add=True)

    def _wait_scatter(j):
        pltpu.make_async_copy(
            rows_v.at[j & 1], acc_sh.at[idx_v.at[j % 3, 1]],
            ssem.at[j & 1]).wait()

    _start_idx(0)

    def _body(j, carry):
        @pl.when(j >= 2)
        def _():
            _wait_scatter(j - 2)

        @pl.when(j + 1 < cnt)
        def _():
            _start_idx(j + 1)

        _wait_idx(j)
        _start_gather(j)

        @pl.when(j >= 1)
        def _():
            _wait_gather(j - 1)
            _start_scatter(j - 1)
        return carry

    lax.fori_loop(0, cnt, _body, 0)
    _wait_gather(cnt - 1)
    _start_scatter(cnt - 1)
    _wait_scatter(cnt - 2)
    _wait_scatter(cnt - 1)
    plsc.subcore_barrier()
    pltpu.sync_copy(
        acc_sh.at[pl.ds(s * ROWS_PER_TILE, ROWS_PER_TILE), :],
        out_hbm.at[c, pl.ds(s * ROWS_PER_TILE, ROWS_PER_TILE), :])


_BLK = 512
_NBLK = PAD_N // _BLK


def _mm_body(deg_ref, x_ref, w_ref, o_ref):
    degs = deg_ref[:, 0:1] + deg_ref[:, 1:2] + 1.0
    dinv = lax.rsqrt(degs)
    h = jnp.dot(x_ref[...], w_ref[...], preferred_element_type=jnp.float32)
    o_ref[...] = h * dinv


def _mm_call(degp_t, x, W):
    return pl.pallas_call(
        _mm_body,
        grid=(_NBLK,),
        in_specs=[
            pl.BlockSpec((_BLK, NC), lambda i: (i, 0)),
            pl.BlockSpec((_BLK, D), lambda i: (i, 0)),
            pl.BlockSpec((D, D), lambda i: (0, 0)),
        ],
        out_specs=pl.BlockSpec((_BLK, D), lambda i: (i, 0)),
        out_shape=jax.ShapeDtypeStruct((N_NODES, D), jnp.float32),
    )(degp_t, x, W)


def _fin_body(deg_ref, s_ref, h_ref, b_ref, a_ref, o_ref):
    degs = deg_ref[:, 0:1] + deg_ref[:, 1:2] + 1.0
    dinv = lax.rsqrt(degs)
    z = (s_ref[0] + s_ref[1] + h_ref[...]) * dinv + b_ref[...]
    o_ref[...] = jnp.where(z >= 0.0, z, a_ref[...] * z)


def _fin_call(degp_t, S, h, b, alpha):
    return pl.pallas_call(
        _fin_body,
        grid=(_NBLK,),
        in_specs=[
            pl.BlockSpec((_BLK, NC), lambda i: (i, 0)),
            pl.BlockSpec((NC, _BLK, D), lambda i: (0, i, 0)),
            pl.BlockSpec((_BLK, D), lambda i: (i, 0)),
            pl.BlockSpec((1, D), lambda i: (0, 0)),
            pl.BlockSpec((1, D), lambda i: (0, 0)),
        ],
        out_specs=pl.BlockSpec((_BLK, D), lambda i: (i, 0)),
        out_shape=jax.ShapeDtypeStruct((N_NODES, D), jnp.float32),
    )(degp_t, S, h, b, alpha)


def kernel(x, edge_index, W, b, alpha):
    ei = edge_index.astype(jnp.int32)
    src = jnp.concatenate(
        [ei[0], jnp.zeros((E_PAD - N_EDGES,), jnp.int32)]
    ).reshape(NCH_TOTAL, 1, CH)
    dst = jnp.concatenate(
        [ei[1], jnp.full((E_PAD - N_EDGES,), PAD_N - 1, jnp.int32)]
    ).reshape(NCH_TOTAL, 1, CH)
    idx = jnp.concatenate([src, dst], axis=1)
    degp = _deg_kernel(idx)
    degp_t = degp.T
    h = _mm_call(degp_t, x, W)
    S = _msg_kernel(idx, h)
    out = _fin_call(degp_t, S, h, b.reshape(1, D), alpha.reshape(1, D))
    return out


# speedup vs baseline: 1.1944x; 1.1944x over previous
import functools

import jax
import jax.numpy as jnp
from jax import lax
from jax.experimental import pallas as pl
from jax.experimental.pallas import tpu as pltpu
from jax.experimental.pallas import tpu_sc as plsc

N_NODES = 10000
D = 128
N_EDGES = 320000
NC, NS = 2, 16
NW = NC * NS
PAD_N = 10240
ROWS_PER_TILE = PAD_N // NS
CH = 128
NCHT = 79
E_PAD = NW * NCHT * CH
NCH_TOTAL = NW * NCHT
NCHT_C0 = 134
NCHT_C1 = 2 * NCHT - NCHT_C0

_mesh = plsc.VectorSubcoreMesh(core_axis_name="c", subcore_axis_name="s")


@functools.partial(
    pl.kernel,
    out_type=jax.ShapeDtypeStruct((NC, PAD_N), jnp.float32),
    mesh=_mesh,
    scratch_types=[
        pltpu.VMEM((NCHT, 2, CH), jnp.int32),
        pltpu.VMEM((CH,), jnp.float32),
        pltpu.VMEM((ROWS_PER_TILE,), jnp.float32),
        pltpu.VMEM_SHARED((PAD_N,), jnp.float32),
        pltpu.SemaphoreType.DMA,
    ],
)
def _deg_kernel(idx_hbm, degp_hbm, idx_v, ones_v, zero_v, acc_sh, sem):
    c = lax.axis_index("c")
    s = lax.axis_index("s")
    w = s * NC + c

    for i in range(CH // 16):
        ones_v[pl.ds(i * 16, 16)] = jnp.ones((16,), jnp.float32)

    def _zfill(i, carry):
        zero_v[pl.ds(i * 16, 16)] = jnp.zeros((16,), jnp.float32)
        return carry

    lax.fori_loop(0, ROWS_PER_TILE // 16, _zfill, 0)
    pltpu.sync_copy(idx_hbm.at[pl.ds(w * NCHT, NCHT)], idx_v)
    pltpu.sync_copy(zero_v, acc_sh.at[pl.ds(s * ROWS_PER_TILE, ROWS_PER_TILE)])
    plsc.subcore_barrier()

    GRP = 16

    def _group(g, carry):
        for k in range(GRP):
            @pl.when(g * GRP + k < NCHT)
            def _():
                pltpu.async_copy(
                    ones_v, acc_sh.at[idx_v.at[g * GRP + k, 1]], sem,
                    add=True)
        for k in range(GRP):
            @pl.when(g * GRP + k < NCHT)
            def _():
                pltpu.make_async_copy(
                    ones_v, acc_sh.at[idx_v.at[g * GRP + k, 1]], sem).wait()
        return carry

    lax.fori_loop(0, (NCHT + GRP - 1) // GRP, _group, 0)
    plsc.subcore_barrier()
    pltpu.sync_copy(acc_sh.at[pl.ds(s * ROWS_PER_TILE, ROWS_PER_TILE)],
                    degp_hbm.at[c, pl.ds(s * ROWS_PER_TILE, ROWS_PER_TILE)])


@functools.partial(
    pl.kernel,
    out_type=jax.ShapeDtypeStruct((NC, PAD_N, D), jnp.float32),
    mesh=_mesh,
    scratch_types=[
        pltpu.VMEM((3, 2, CH), jnp.int32),
        pltpu.VMEM((2, CH, D), jnp.float32),
        pltpu.VMEM((16, D), jnp.float32),
        pltpu.VMEM_SHARED((PAD_N, D), jnp.float32),
        pltpu.SemaphoreType.DMA((3,)),
        pltpu.SemaphoreType.DMA((2,)),
        pltpu.SemaphoreType.DMA((2,)),
    ],
)
def _msg_kernel(idx_hbm, h_hbm, out_hbm,
                idx_v, rows_v, zero_v, acc_sh, isem, gsem, ssem):
    c = lax.axis_index("c")
    s = lax.axis_index("s")
    start = jnp.where(c == 0, s * NCHT_C0, NS * NCHT_C0 + s * NCHT_C1)
    cnt = jnp.where(c == 0, NCHT_C0, NCHT_C1)

    for r in range(16):
        for k in range(D // 16):
            zero_v[r, pl.ds(k * 16, 16)] = jnp.zeros((16,), jnp.float32)

    def _zfill(i, carry):
        pltpu.sync_copy(
            zero_v, acc_sh.at[pl.ds(s * ROWS_PER_TILE + i * 16, 16), :])
        return carry

    lax.fori_loop(0, ROWS_PER_TILE // 16, _zfill, 0)
    plsc.subcore_barrier()

    def _start_idx(j):
        pltpu.async_copy(idx_hbm.at[start + j], idx_v.at[j % 3],
                         isem.at[j % 3])

    def _wait_idx(j):
        pltpu.make_async_copy(
            idx_hbm.at[start + j], idx_v.at[j % 3], isem.at[j % 3]).wait()

    def _start_gather(j):
        pltpu.async_copy(
            h_hbm.at[idx_v.at[j % 3, 0]], rows_v.at[j & 1], gsem.at[j & 1])

    def _wait_gather(j):
        pltpu.make_async_copy(
            h_hbm.at[idx_v.at[j % 3, 0]], rows_v.at[j & 1],
            gsem.at[j & 1]).wait()

    def _start_scatter(j):
        pltpu.async_copy(
            rows_v.at[j & 1], acc_sh.at[idx_v.at[j % 3, 1]], ssem.at[j & 1],
            add=True)

    def _wait_scatter(j):
        pltpu.make_async_copy(
            rows_v.at[j & 1], acc_sh.at[idx_v.at[j % 3, 1]],
            ssem.at[j & 1]).wait()

    _start_idx(0)

    def _body(j, carry):
        @pl.when(j >= 2)
        def _():
            _wait_scatter(j - 2)

        @pl.when(j + 1 < cnt)
        def _():
            _start_idx(j + 1)

        _wait_idx(j)
        _start_gather(j)

        @pl.when(j >= 1)
        def _():
            _wait_gather(j - 1)
            _start_scatter(j - 1)
        return carry

    lax.fori_loop(0, cnt, _body, 0)
    _wait_gather(cnt - 1)
    _start_scatter(cnt - 1)
    _wait_scatter(cnt - 2)
    _wait_scatter(cnt - 1)
    plsc.subcore_barrier()
    pltpu.sync_copy(
        acc_sh.at[pl.ds(s * ROWS_PER_TILE, ROWS_PER_TILE), :],
        out_hbm.at[c, pl.ds(s * ROWS_PER_TILE, ROWS_PER_TILE), :])


_BLK = 512
_NBLK = PAD_N // _BLK


def _mm_body(deg_ref, x_ref, w_ref, o_ref):
    degs = deg_ref[:, 0:1] + deg_ref[:, 1:2] + 1.0
    dinv = lax.rsqrt(degs)
    h = jnp.dot(x_ref[...], w_ref[...], preferred_element_type=jnp.float32)
    o_ref[...] = h * dinv


def _mm_call(degp_t, x, W):
    return pl.pallas_call(
        _mm_body,
        grid=(_NBLK,),
        in_specs=[
            pl.BlockSpec((_BLK, NC), lambda i: (i, 0)),
            pl.BlockSpec((_BLK, D), lambda i: (i, 0)),
            pl.BlockSpec((D, D), lambda i: (0, 0)),
        ],
        out_specs=pl.BlockSpec((_BLK, D), lambda i: (i, 0)),
        out_shape=jax.ShapeDtypeStruct((N_NODES, D), jnp.float32),
    )(degp_t, x, W)


def _fin_body(deg_ref, s_ref, h_ref, b_ref, a_ref, o_ref):
    degs = deg_ref[:, 0:1] + deg_ref[:, 1:2] + 1.0
    dinv = lax.rsqrt(degs)
    z = (s_ref[0] + s_ref[1] + h_ref[...]) * dinv + b_ref[...]
    o_ref[...] = jnp.where(z >= 0.0, z, a_ref[...] * z)


def _fin_call(degp_t, S, h, b, alpha):
    return pl.pallas_call(
        _fin_body,
        grid=(_NBLK,),
        in_specs=[
            pl.BlockSpec((_BLK, NC), lambda i: (i, 0)),
            pl.BlockSpec((NC, _BLK, D), lambda i: (0, i, 0)),
            pl.BlockSpec((_BLK, D), lambda i: (i, 0)),
            pl.BlockSpec((1, D), lambda i: (0, 0)),
            pl.BlockSpec((1, D), lambda i: (0, 0)),
        ],
        out_specs=pl.BlockSpec((_BLK, D), lambda i: (i, 0)),
        out_shape=jax.ShapeDtypeStruct((N_NODES, D), jnp.float32),
    )(degp_t, S, h, b, alpha)


def kernel(x, edge_index, W, b, alpha):
    ei = edge_index.astype(jnp.int32)
    src = jnp.concatenate(
        [ei[0], jnp.zeros((E_PAD - N_EDGES,), jnp.int32)]
    ).reshape(NCH_TOTAL, 1, CH)
    dst = jnp.concatenate(
        [ei[1], jnp.full((E_PAD - N_EDGES,), PAD_N - 1, jnp.int32)]
    ).reshape(NCH_TOTAL, 1, CH)
    idx = jnp.concatenate([src, dst], axis=1)
    degp = _deg_kernel(idx)
    degp_t = degp.T
    h = _mm_call(degp_t, x, W)
    S = _msg_kernel(idx, h)
    out = _fin_call(degp_t, S, h, b.reshape(1, D), alpha.reshape(1, D))
    return out


# speedup vs baseline: 1.2040x; 1.0081x over previous
import functools

import jax
import jax.numpy as jnp
from jax import lax
from jax.experimental import pallas as pl
from jax.experimental.pallas import tpu as pltpu
from jax.experimental.pallas import tpu_sc as plsc

N_NODES = 10000
D = 128
N_EDGES = 320000
NC, NS = 2, 16
NW = NC * NS
PAD_N = 10240
ROWS_PER_TILE = PAD_N // NS
CH = 128
NCHT = 79
E_PAD = NW * NCHT * CH
NCH_TOTAL = NW * NCHT
NCHT_C0 = 140
NCHT_C1 = 2 * NCHT - NCHT_C0

_mesh = plsc.VectorSubcoreMesh(core_axis_name="c", subcore_axis_name="s")


@functools.partial(
    pl.kernel,
    out_type=jax.ShapeDtypeStruct((NC, PAD_N), jnp.float32),
    mesh=_mesh,
    scratch_types=[
        pltpu.VMEM((NCHT, 2, CH), jnp.int32),
        pltpu.VMEM((CH,), jnp.float32),
        pltpu.VMEM((ROWS_PER_TILE,), jnp.float32),
        pltpu.VMEM_SHARED((PAD_N,), jnp.float32),
        pltpu.SemaphoreType.DMA,
    ],
)
def _deg_kernel(idx_hbm, degp_hbm, idx_v, ones_v, zero_v, acc_sh, sem):
    c = lax.axis_index("c")
    s = lax.axis_index("s")
    w = s * NC + c

    for i in range(CH // 16):
        ones_v[pl.ds(i * 16, 16)] = jnp.ones((16,), jnp.float32)

    def _zfill(i, carry):
        zero_v[pl.ds(i * 16, 16)] = jnp.zeros((16,), jnp.float32)
        return carry

    lax.fori_loop(0, ROWS_PER_TILE // 16, _zfill, 0)
    pltpu.sync_copy(idx_hbm.at[pl.ds(w * NCHT, NCHT)], idx_v)
    pltpu.sync_copy(zero_v, acc_sh.at[pl.ds(s * ROWS_PER_TILE, ROWS_PER_TILE)])
    plsc.subcore_barrier()

    GRP = 16

    def _group(g, carry):
        for k in range(GRP):
            @pl.when(g * GRP + k < NCHT)
            def _():
                pltpu.async_copy(
                    ones_v, acc_sh.at[idx_v.at[g * GRP + k, 1]], sem,
                    add=True)
        for k in range(GRP):
            @pl.when(g * GRP + k < NCHT)
            def _():
                pltpu.make_async_copy(
                    ones_v, acc_sh.at[idx_v.at[g * GRP + k, 1]], sem).wait()
        return carry

    lax.fori_loop(0, (NCHT + GRP - 1) // GRP, _group, 0)
    plsc.subcore_barrier()
    pltpu.sync_copy(acc_sh.at[pl.ds(s * ROWS_PER_TILE, ROWS_PER_TILE)],
                    degp_hbm.at[c, pl.ds(s * ROWS_PER_TILE, ROWS_PER_TILE)])


@functools.partial(
    pl.kernel,
    out_type=jax.ShapeDtypeStruct((NC, PAD_N, D), jnp.float32),
    mesh=_mesh,
    scratch_types=[
        pltpu.VMEM((3, 2, CH), jnp.int32),
        pltpu.VMEM((2, CH, D), jnp.float32),
        pltpu.VMEM((16, D), jnp.float32),
        pltpu.VMEM_SHARED((PAD_N, D), jnp.float32),
        pltpu.SemaphoreType.DMA((3,)),
        pltpu.SemaphoreType.DMA((2,)),
        pltpu.SemaphoreType.DMA((2,)),
        pltpu.SemaphoreType.DMA,
    ],
)
def _msg_kernel(idx_hbm, h_hbm, out_hbm,
                idx_v, rows_v, zero_v, acc_sh, isem, gsem, ssem, zsem):
    c = lax.axis_index("c")
    s = lax.axis_index("s")
    start = jnp.where(c == 0, s * NCHT_C0, NS * NCHT_C0 + s * NCHT_C1)
    cnt = jnp.where(c == 0, NCHT_C0, NCHT_C1)

    for r in range(16):
        for k in range(D // 16):
            zero_v[r, pl.ds(k * 16, 16)] = jnp.zeros((16,), jnp.float32)

    def _zfire(i, carry):
        pltpu.async_copy(
            zero_v, acc_sh.at[pl.ds(s * ROWS_PER_TILE + i * 16, 16), :],
            zsem)
        return carry

    lax.fori_loop(0, ROWS_PER_TILE // 16, _zfire, 0)

    def _zdrain(i, carry):
        pltpu.make_async_copy(
            zero_v, acc_sh.at[pl.ds(s * ROWS_PER_TILE + i * 16, 16), :],
            zsem).wait()
        return carry

    lax.fori_loop(0, ROWS_PER_TILE // 16, _zdrain, 0)
    plsc.subcore_barrier()

    def _start_idx(j):
        pltpu.async_copy(idx_hbm.at[start + j], idx_v.at[j % 3],
                         isem.at[j % 3])

    def _wait_idx(j):
        pltpu.make_async_copy(
            idx_hbm.at[start + j], idx_v.at[j % 3], isem.at[j % 3]).wait()

    def _start_gather(j):
        pltpu.async_copy(
            h_hbm.at[idx_v.at[j % 3, 0]], rows_v.at[j & 1], gsem.at[j & 1])

    def _wait_gather(j):
        pltpu.make_async_copy(
            h_hbm.at[idx_v.at[j % 3, 0]], rows_v.at[j & 1],
            gsem.at[j & 1]).wait()

    def _start_scatter(j):
        pltpu.async_copy(
            rows_v.at[j & 1], acc_sh.at[idx_v.at[j % 3, 1]], ssem.at[j & 1],
            add=True)

    def _wait_scatter(j):
        pltpu.make_async_copy(
            rows_v.at[j & 1], acc_sh.at[idx_v.at[j % 3, 1]],
            ssem.at[j & 1]).wait()

    _start_idx(0)

    def _body(j, carry):
        @pl.when(j >= 2)
        def _():
            _wait_scatter(j - 2)

        @pl.when(j + 1 < cnt)
        def _():
            _start_idx(j + 1)

        _wait_idx(j)
        _start_gather(j)

        @pl.when(j >= 1)
        def _():
            _wait_gather(j - 1)
            _start_scatter(j - 1)
        return carry

    lax.fori_loop(0, cnt, _body, 0)
    _wait_gather(cnt - 1)
    _start_scatter(cnt - 1)
    _wait_scatter(cnt - 2)
    _wait_scatter(cnt - 1)
    plsc.subcore_barrier()
    pltpu.sync_copy(
        acc_sh.at[pl.ds(s * ROWS_PER_TILE, ROWS_PER_TILE), :],
        out_hbm.at[c, pl.ds(s * ROWS_PER_TILE, ROWS_PER_TILE), :])


_BLK = 512
_NBLK = PAD_N // _BLK


def _mm_body(deg_ref, x_ref, w_ref, o_ref):
    degs = deg_ref[:, 0:1] + deg_ref[:, 1:2] + 1.0
    dinv = lax.rsqrt(degs)
    h = jnp.dot(x_ref[...], w_ref[...], preferred_element_type=jnp.float32)
    o_ref[...] = h * dinv


def _mm_call(degp_t, x, W):
    return pl.pallas_call(
        _mm_body,
        grid=(_NBLK,),
        in_specs=[
            pl.BlockSpec((_BLK, NC), lambda i: (i, 0)),
            pl.BlockSpec((_BLK, D), lambda i: (i, 0)),
            pl.BlockSpec((D, D), lambda i: (0, 0)),
        ],
        out_specs=pl.BlockSpec((_BLK, D), lambda i: (i, 0)),
        out_shape=jax.ShapeDtypeStruct((N_NODES, D), jnp.float32),
    )(degp_t, x, W)


def _fin_body(deg_ref, s_ref, h_ref, b_ref, a_ref, o_ref):
    degs = deg_ref[:, 0:1] + deg_ref[:, 1:2] + 1.0
    dinv = lax.rsqrt(degs)
    z = (s_ref[0] + s_ref[1] + h_ref[...]) * dinv + b_ref[...]
    o_ref[...] = jnp.where(z >= 0.0, z, a_ref[...] * z)


def _fin_call(degp_t, S, h, b, alpha):
    return pl.pallas_call(
        _fin_body,
        grid=(_NBLK,),
        in_specs=[
            pl.BlockSpec((_BLK, NC), lambda i: (i, 0)),
            pl.BlockSpec((NC, _BLK, D), lambda i: (0, i, 0)),
            pl.BlockSpec((_BLK, D), lambda i: (i, 0)),
            pl.BlockSpec((1, D), lambda i: (0, 0)),
            pl.BlockSpec((1, D), lambda i: (0, 0)),
        ],
        out_specs=pl.BlockSpec((_BLK, D), lambda i: (i, 0)),
        out_shape=jax.ShapeDtypeStruct((N_NODES, D), jnp.float32),
    )(degp_t, S, h, b, alpha)


def kernel(x, edge_index, W, b, alpha):
    ei = edge_index.astype(jnp.int32)
    src = jnp.concatenate(
        [ei[0], jnp.zeros((E_PAD - N_EDGES,), jnp.int32)]
    ).reshape(NCH_TOTAL, 1, CH)
    dst = jnp.concatenate(
        [ei[1], jnp.full((E_PAD - N_EDGES,), PAD_N - 1, jnp.int32)]
    ).reshape(NCH_TOTAL, 1, CH)
    idx = jnp.concatenate([src, dst], axis=1)
    degp = _deg_kernel(idx)
    degp_t = degp.T
    h = _mm_call(degp_t, x, W)
    S = _msg_kernel(idx, h)
    out = _fin_call(degp_t, S, h, b.reshape(1, D), alpha.reshape(1, D))
    return out


# speedup vs baseline: 1.2059x; 1.0016x over previous
import functools

import jax
import jax.numpy as jnp
from jax import lax
from jax.experimental import pallas as pl
from jax.experimental.pallas import tpu as pltpu
from jax.experimental.pallas import tpu_sc as plsc

N_NODES = 10000
D = 128
N_EDGES = 320000
NC, NS = 2, 16
NW = NC * NS
PAD_N = 10240
ROWS_PER_TILE = PAD_N // NS
CH = 128
NCHT = 79
E_PAD = NW * NCHT * CH
NCH_TOTAL = NW * NCHT
NCHT_C0 = 140
NCHT_C1 = 2 * NCHT - NCHT_C0

_mesh = plsc.VectorSubcoreMesh(core_axis_name="c", subcore_axis_name="s")


@functools.partial(
    pl.kernel,
    out_type=jax.ShapeDtypeStruct((NC, PAD_N), jnp.float32),
    mesh=_mesh,
    scratch_types=[
        pltpu.VMEM((NCHT, 2, CH), jnp.int32),
        pltpu.VMEM((CH,), jnp.float32),
        pltpu.VMEM((ROWS_PER_TILE,), jnp.float32),
        pltpu.VMEM_SHARED((PAD_N,), jnp.float32),
        pltpu.SemaphoreType.DMA,
    ],
)
def _deg_kernel(idx_hbm, degp_hbm, idx_v, ones_v, zero_v, acc_sh, sem):
    c = lax.axis_index("c")
    s = lax.axis_index("s")
    w = s * NC + c

    for i in range(CH // 16):
        ones_v[pl.ds(i * 16, 16)] = jnp.ones((16,), jnp.float32)

    def _zfill(i, carry):
        zero_v[pl.ds(i * 16, 16)] = jnp.zeros((16,), jnp.float32)
        return carry

    lax.fori_loop(0, ROWS_PER_TILE // 16, _zfill, 0)
    pltpu.sync_copy(idx_hbm.at[pl.ds(w * NCHT, NCHT)], idx_v)
    pltpu.sync_copy(zero_v, acc_sh.at[pl.ds(s * ROWS_PER_TILE, ROWS_PER_TILE)])
    plsc.subcore_barrier()

    GRP = 16

    def _group(g, carry):
        for k in range(GRP):
            @pl.when(g * GRP + k < NCHT)
            def _():
                pltpu.async_copy(
                    ones_v, acc_sh.at[idx_v.at[g * GRP + k, 1]], sem,
                    add=True)
        for k in range(GRP):
            @pl.when(g * GRP + k < NCHT)
            def _():
                pltpu.make_async_copy(
                    ones_v, acc_sh.at[idx_v.at[g * GRP + k, 1]], sem).wait()
        return carry

    lax.fori_loop(0, (NCHT + GRP - 1) // GRP, _group, 0)
    plsc.subcore_barrier()
    pltpu.sync_copy(acc_sh.at[pl.ds(s * ROWS_PER_TILE, ROWS_PER_TILE)],
                    degp_hbm.at[c, pl.ds(s * ROWS_PER_TILE, ROWS_PER_TILE)])


@functools.partial(
    pl.kernel,
    out_type=jax.ShapeDtypeStruct((NC, PAD_N, D), jnp.float32),
    mesh=_mesh,
    scratch_types=[
        pltpu.VMEM((3, 2, CH), jnp.int32),
        pltpu.VMEM((2, CH, D), jnp.float32),
        pltpu.VMEM((16, D), jnp.float32),
        pltpu.VMEM_SHARED((PAD_N, D), jnp.float32),
        pltpu.SemaphoreType.DMA((3,)),
        pltpu.SemaphoreType.DMA((2,)),
        pltpu.SemaphoreType.DMA((2,)),
        pltpu.SemaphoreType.DMA,
    ],
)
def _msg_kernel(idx_hbm, h_hbm, out_hbm,
                idx_v, rows_v, zero_v, acc_sh, isem, gsem, ssem, zsem):
    c = lax.axis_index("c")
    s = lax.axis_index("s")
    start = jnp.where(c == 0, s * NCHT_C0, NS * NCHT_C0 + s * NCHT_C1)
    cnt = jnp.where(c == 0, NCHT_C0, NCHT_C1)

    for r in range(16):
        for k in range(D // 16):
            zero_v[r, pl.ds(k * 16, 16)] = jnp.zeros((16,), jnp.float32)

    def _zfire(i, carry):
        pltpu.async_copy(
            zero_v, acc_sh.at[pl.ds(s * ROWS_PER_TILE + i * 16, 16), :],
            zsem)
        return carry

    lax.fori_loop(0, ROWS_PER_TILE // 16, _zfire, 0)
    pltpu.async_copy(idx_hbm.at[start], idx_v.at[0], isem.at[0])

    def _zdrain(i, carry):
        pltpu.make_async_copy(
            zero_v, acc_sh.at[pl.ds(s * ROWS_PER_TILE + i * 16, 16), :],
            zsem).wait()
        return carry

    lax.fori_loop(0, ROWS_PER_TILE // 16, _zdrain, 0)
    plsc.subcore_barrier()

    def _start_idx(j):
        pltpu.async_copy(idx_hbm.at[start + j], idx_v.at[j % 3],
                         isem.at[j % 3])

    def _wait_idx(j):
        pltpu.make_async_copy(
            idx_hbm.at[start + j], idx_v.at[j % 3], isem.at[j % 3]).wait()

    def _start_gather(j):
        pltpu.async_copy(
            h_hbm.at[idx_v.at[j % 3, 0]], rows_v.at[j & 1], gsem.at[j & 1])

    def _wait_gather(j):
        pltpu.make_async_copy(
            h_hbm.at[idx_v.at[j % 3, 0]], rows_v.at[j & 1],
            gsem.at[j & 1]).wait()

    def _start_scatter(j):
        pltpu.async_copy(
            rows_v.at[j & 1], acc_sh.at[idx_v.at[j % 3, 1]], ssem.at[j & 1],
            add=True)

    def _wait_scatter(j):
        pltpu.make_async_copy(
            rows_v.at[j & 1], acc_sh.at[idx_v.at[j % 3, 1]],
            ssem.at[j & 1]).wait()

    def _body(j, carry):
        @pl.when(j >= 2)
        def _():
            _wait_scatter(j - 2)

        @pl.when(j + 1 < cnt)
        def _():
            _start_idx(j + 1)

        _wait_idx(j)
        _start_gather(j)

        @pl.when(j >= 1)
        def _():
            _wait_gather(j - 1)
            _start_scatter(j - 1)
        return carry

    lax.fori_loop(0, cnt, _body, 0)
    _wait_gather(cnt - 1)
    _start_scatter(cnt - 1)
    _wait_scatter(cnt - 2)
    _wait_scatter(cnt - 1)
    plsc.subcore_barrier()
    pltpu.sync_copy(
        acc_sh.at[pl.ds(s * ROWS_PER_TILE, ROWS_PER_TILE), :],
        out_hbm.at[c, pl.ds(s * ROWS_PER_TILE, ROWS_PER_TILE), :])


_BLK = 512
_NBLK = PAD_N // _BLK


def _mm_body(deg_ref, x_ref, w_ref, o_ref):
    degs = deg_ref[:, 0:1] + deg_ref[:, 1:2] + 1.0
    dinv = lax.rsqrt(degs)
    h = jnp.dot(x_ref[...], w_ref[...], preferred_element_type=jnp.float32)
    o_ref[...] = h * dinv


def _mm_call(degp_t, x, W):
    return pl.pallas_call(
        _mm_body,
        grid=(_NBLK,),
        in_specs=[
            pl.BlockSpec((_BLK, NC), lambda i: (i, 0)),
            pl.BlockSpec((_BLK, D), lambda i: (i, 0)),
            pl.BlockSpec((D, D), lambda i: (0, 0)),
        ],
        out_specs=pl.BlockSpec((_BLK, D), lambda i: (i, 0)),
        out_shape=jax.ShapeDtypeStruct((N_NODES, D), jnp.float32),
    )(degp_t, x, W)


def _fin_body(deg_ref, s_ref, h_ref, b_ref, a_ref, o_ref):
    degs = deg_ref[:, 0:1] + deg_ref[:, 1:2] + 1.0
    dinv = lax.rsqrt(degs)
    z = (s_ref[0] + s_ref[1] + h_ref[...]) * dinv + b_ref[...]
    o_ref[...] = jnp.where(z >= 0.0, z, a_ref[...] * z)


def _fin_call(degp_t, S, h, b, alpha):
    return pl.pallas_call(
        _fin_body,
        grid=(_NBLK,),
        in_specs=[
            pl.BlockSpec((_BLK, NC), lambda i: (i, 0)),
            pl.BlockSpec((NC, _BLK, D), lambda i: (0, i, 0)),
            pl.BlockSpec((_BLK, D), lambda i: (i, 0)),
            pl.BlockSpec((1, D), lambda i: (0, 0)),
            pl.BlockSpec((1, D), lambda i: (0, 0)),
        ],
        out_specs=pl.BlockSpec((_BLK, D), lambda i: (i, 0)),
        out_shape=jax.ShapeDtypeStruct((N_NODES, D), jnp.float32),
    )(degp_t, S, h, b, alpha)


def kernel(x, edge_index, W, b, alpha):
    ei = edge_index.astype(jnp.int32)
    src = jnp.concatenate(
        [ei[0], jnp.zeros((E_PAD - N_EDGES,), jnp.int32)]
    ).reshape(NCH_TOTAL, 1, CH)
    dst = jnp.concatenate(
        [ei[1], jnp.full((E_PAD - N_EDGES,), PAD_N - 1, jnp.int32)]
    ).reshape(NCH_TOTAL, 1, CH)
    idx = jnp.concatenate([src, dst], axis=1)
    degp = _deg_kernel(idx)
    degp_t = degp.T
    h = _mm_call(degp_t, x, W)
    S = _msg_kernel(idx, h)
    out = _fin_call(degp_t, S, h, b.reshape(1, D), alpha.reshape(1, D))
    return out


# speedup vs baseline: 1.2067x; 1.0007x over previous
import functools

import jax
import jax.numpy as jnp
from jax import lax
from jax.experimental import pallas as pl
from jax.experimental.pallas import tpu as pltpu
from jax.experimental.pallas import tpu_sc as plsc

N_NODES = 10000
D = 128
N_EDGES = 320000
NC, NS = 2, 16
NW = NC * NS
PAD_N = 10240
ROWS_PER_TILE = PAD_N // NS
CH = 128
NCHT = 79
E_PAD = NW * NCHT * CH
NCH_TOTAL = NW * NCHT
NCHT_C0 = 140
NCHT_C1 = 2 * NCHT - NCHT_C0

_mesh = plsc.VectorSubcoreMesh(core_axis_name="c", subcore_axis_name="s")


@functools.partial(
    pl.kernel,
    out_type=jax.ShapeDtypeStruct((NC, PAD_N), jnp.float32),
    mesh=_mesh,
    scratch_types=[
        pltpu.VMEM((NCHT, 2, CH), jnp.int32),
        pltpu.VMEM((CH,), jnp.float32),
        pltpu.VMEM((ROWS_PER_TILE,), jnp.float32),
        pltpu.VMEM_SHARED((PAD_N,), jnp.float32),
        pltpu.SemaphoreType.DMA,
    ],
)
def _deg_kernel(idx_hbm, degp_hbm, idx_v, ones_v, zero_v, acc_sh, sem):
    c = lax.axis_index("c")
    s = lax.axis_index("s")
    w = s * NC + c

    for i in range(CH // 16):
        ones_v[pl.ds(i * 16, 16)] = jnp.ones((16,), jnp.float32)

    def _zfill(i, carry):
        zero_v[pl.ds(i * 16, 16)] = jnp.zeros((16,), jnp.float32)
        return carry

    lax.fori_loop(0, ROWS_PER_TILE // 16, _zfill, 0)
    pltpu.sync_copy(idx_hbm.at[pl.ds(w * NCHT, NCHT)], idx_v)
    pltpu.sync_copy(zero_v, acc_sh.at[pl.ds(s * ROWS_PER_TILE, ROWS_PER_TILE)])
    plsc.subcore_barrier()

    GRP = 16
    NGRP = (NCHT + GRP - 1) // GRP

    def _fire(g):
        for k in range(GRP):
            @pl.when(g * GRP + k < NCHT)
            def _():
                pltpu.async_copy(
                    ones_v, acc_sh.at[idx_v.at[g * GRP + k, 1]], sem,
                    add=True)

    def _drain(g):
        for k in range(GRP):
            @pl.when(g * GRP + k < NCHT)
            def _():
                pltpu.make_async_copy(
                    ones_v, acc_sh.at[idx_v.at[g * GRP + k, 1]], sem).wait()

    _fire(0)

    def _group(g, carry):
        _fire(g)
        _drain(g - 1)
        return carry

    lax.fori_loop(1, NGRP, _group, 0)
    _drain(NGRP - 1)
    plsc.subcore_barrier()
    pltpu.sync_copy(acc_sh.at[pl.ds(s * ROWS_PER_TILE, ROWS_PER_TILE)],
                    degp_hbm.at[c, pl.ds(s * ROWS_PER_TILE, ROWS_PER_TILE)])


@functools.partial(
    pl.kernel,
    out_type=jax.ShapeDtypeStruct((NC, PAD_N, D), jnp.float32),
    mesh=_mesh,
    scratch_types=[
        pltpu.VMEM((3, 2, CH), jnp.int32),
        pltpu.VMEM((2, CH, D), jnp.float32),
        pltpu.VMEM((16, D), jnp.float32),
        pltpu.VMEM_SHARED((PAD_N, D), jnp.float32),
        pltpu.SemaphoreType.DMA((3,)),
        pltpu.SemaphoreType.DMA((2,)),
        pltpu.SemaphoreType.DMA((2,)),
        pltpu.SemaphoreType.DMA,
    ],
)
def _msg_kernel(idx_hbm, h_hbm, out_hbm,
                idx_v, rows_v, zero_v, acc_sh, isem, gsem, ssem, zsem):
    c = lax.axis_index("c")
    s = lax.axis_index("s")
    start = jnp.where(c == 0, s * NCHT_C0, NS * NCHT_C0 + s * NCHT_C1)
    cnt = jnp.where(c == 0, NCHT_C0, NCHT_C1)

    for r in range(16):
        for k in range(D // 16):
            zero_v[r, pl.ds(k * 16, 16)] = jnp.zeros((16,), jnp.float32)

    def _zfire(i, carry):
        pltpu.async_copy(
            zero_v, acc_sh.at[pl.ds(s * ROWS_PER_TILE + i * 16, 16), :],
            zsem)
        return carry

    lax.fori_loop(0, ROWS_PER_TILE // 16, _zfire, 0)
    pltpu.async_copy(idx_hbm.at[start], idx_v.at[0], isem.at[0])

    def _zdrain(i, carry):
        pltpu.make_async_copy(
            zero_v, acc_sh.at[pl.ds(s * ROWS_PER_TILE + i * 16, 16), :],
            zsem).wait()
        return carry

    lax.fori_loop(0, ROWS_PER_TILE // 16, _zdrain, 0)
    plsc.subcore_barrier()

    def _start_idx(j):
        pltpu.async_copy(idx_hbm.at[start + j], idx_v.at[j % 3],
                         isem.at[j % 3])

    def _wait_idx(j):
        pltpu.make_async_copy(
            idx_hbm.at[start + j], idx_v.at[j % 3], isem.at[j % 3]).wait()

    def _start_gather(j):
        pltpu.async_copy(
            h_hbm.at[idx_v.at[j % 3, 0]], rows_v.at[j & 1], gsem.at[j & 1])

    def _wait_gather(j):
        pltpu.make_async_copy(
            h_hbm.at[idx_v.at[j % 3, 0]], rows_v.at[j & 1],
            gsem.at[j & 1]).wait()

    def _start_scatter(j):
        pltpu.async_copy(
            rows_v.at[j & 1], acc_sh.at[idx_v.at[j % 3, 1]], ssem.at[j & 1],
            add=True)

    def _wait_scatter(j):
        pltpu.make_async_copy(
            rows_v.at[j & 1], acc_sh.at[idx_v.at[j % 3, 1]],
            ssem.at[j & 1]).wait()

    def _body(j, carry):
        @pl.when(j >= 2)
        def _():
            _wait_scatter(j - 2)

        @pl.when(j + 1 < cnt)
        def _():
            _start_idx(j + 1)

        _wait_idx(j)
        _start_gather(j)

        @pl.when(j >= 1)
        def _():
            _wait_gather(j - 1)
            _start_scatter(j - 1)
        return carry

    lax.fori_loop(0, cnt, _body, 0)
    _wait_gather(cnt - 1)
    _start_scatter(cnt - 1)
    _wait_scatter(cnt - 2)
    _wait_scatter(cnt - 1)
    plsc.subcore_barrier()
    pltpu.sync_copy(
        acc_sh.at[pl.ds(s * ROWS_PER_TILE, ROWS_PER_TILE), :],
        out_hbm.at[c, pl.ds(s * ROWS_PER_TILE, ROWS_PER_TILE), :])


_BLK = 512
_NBLK = PAD_N // _BLK


def _mm_body(deg_ref, x_ref, w_ref, o_ref):
    degs = deg_ref[:, 0:1] + deg_ref[:, 1:2] + 1.0
    dinv = lax.rsqrt(degs)
    h = jnp.dot(x_ref[...], w_ref[...], preferred_element_type=jnp.float32)
    o_ref[...] = h * dinv


def _mm_call(degp_t, x, W):
    return pl.pallas_call(
        _mm_body,
        grid=(_NBLK,),
        in_specs=[
            pl.BlockSpec((_BLK, NC), lambda i: (i, 0)),
            pl.BlockSpec((_BLK, D), lambda i: (i, 0)),
            pl.BlockSpec((D, D), lambda i: (0, 0)),
        ],
        out_specs=pl.BlockSpec((_BLK, D), lambda i: (i, 0)),
        out_shape=jax.ShapeDtypeStruct((N_NODES, D), jnp.float32),
    )(degp_t, x, W)


def _fin_body(deg_ref, s_ref, h_ref, b_ref, a_ref, o_ref):
    degs = deg_ref[:, 0:1] + deg_ref[:, 1:2] + 1.0
    dinv = lax.rsqrt(degs)
    z = (s_ref[0] + s_ref[1] + h_ref[...]) * dinv + b_ref[...]
    o_ref[...] = jnp.where(z >= 0.0, z, a_ref[...] * z)


def _fin_call(degp_t, S, h, b, alpha):
    return pl.pallas_call(
        _fin_body,
        grid=(_NBLK,),
        in_specs=[
            pl.BlockSpec((_BLK, NC), lambda i: (i, 0)),
            pl.BlockSpec((NC, _BLK, D), lambda i: (0, i, 0)),
            pl.BlockSpec((_BLK, D), lambda i: (i, 0)),
            pl.BlockSpec((1, D), lambda i: (0, 0)),
            pl.BlockSpec((1, D), lambda i: (0, 0)),
        ],
        out_specs=pl.BlockSpec((_BLK, D), lambda i: (i, 0)),
        out_shape=jax.ShapeDtypeStruct((N_NODES, D), jnp.float32),
    )(degp_t, S, h, b, alpha)


def kernel(x, edge_index, W, b, alpha):
    ei = edge_index.astype(jnp.int32)
    src = jnp.concatenate(
        [ei[0], jnp.zeros((E_PAD - N_EDGES,), jnp.int32)]
    ).reshape(NCH_TOTAL, 1, CH)
    dst = jnp.concatenate(
        [ei[1], jnp.full((E_PAD - N_EDGES,), PAD_N - 1, jnp.int32)]
    ).reshape(NCH_TOTAL, 1, CH)
    idx = jnp.concatenate([src, dst], axis=1)
    degp = _deg_kernel(idx)
    degp_t = degp.T
    h = _mm_call(degp_t, x, W)
    S = _msg_kernel(idx, h)
    out = _fin_call(degp_t, S, h, b.reshape(1, D), alpha.reshape(1, D))
    return out
